# Initial kernel scaffold; baseline (speedup 1.0000x reference)
import jax
import jax.numpy as jnp
from jax.experimental import pallas as pl


def kernel(x0, x1, x2, up0_index, up1_index, down1_index, down2_index, batch0, batch1, batch2, Wup, bup, Wdn, bdn, Wu1, bu1, Wu2, bu2, Wf1, bf1, Wf2, bf2, Wph, bph, Wl1, bl1, Wl2, bl2):
    raise NotImplementedError("write your pallas kernel here")


# speedup vs baseline: 1.7855x; 1.7855x over previous
import jax
import jax.numpy as jnp
from jax.experimental import pallas as pl

L = 3
N0, N1, N2 = 10000, 160000, 40000
D = 128
B = 64
NC_OUT = 10


def _msg_agg(x, eidx, W, b, n):
    src, dst = eidx[0], eidx[1]
    a = x @ W[:D]
    c = x @ W[D:] + b
    m = jax.nn.relu(a[src] + c[dst])
    return jax.ops.segment_sum(m, dst, num_segments=n)


def _update(x, W1, b1, W2, b2):
    h = jax.nn.relu(x @ W1 + b1)
    return jax.nn.relu(h @ W2 + b2)


def _seg_max(v, seg, n):
    m = jax.ops.segment_max(v, seg, num_segments=n)
    return jnp.where(jnp.isfinite(m), m, 0.0)


def _head_kernel(pooled_ref, ph_ref, Wl1_ref, bl1_ref, Wl2_ref, bl2_ref, o_ref):
    h = jax.nn.relu(jnp.dot(pooled_ref[...], Wl1_ref[...],
                            preferred_element_type=jnp.float32) + bl1_ref[...])
    hcat = jnp.concatenate([h, ph_ref[...]], axis=1)
    o_ref[...] = jnp.dot(hcat, Wl2_ref[...],
                         preferred_element_type=jnp.float32) + bl2_ref[...]


def kernel(x0, x1, x2, up0_index, up1_index, down1_index, down2_index,
           batch0, batch1, batch2,
           Wup, bup, Wdn, bdn, Wu1, bu1, Wu2, bu2,
           Wf1, bf1, Wf2, bf2, Wph, bph, Wl1, bl1, Wl2, bl2):
    ph_vecs = []
    for l in range(L):
        up0 = _msg_agg(x0, up0_index, Wup[l], bup[l], N0)
        up1 = _msg_agg(x1, up1_index, Wup[l], bup[l], N1)
        dn1 = _msg_agg(x1, down1_index, Wdn[l], bdn[l], N1)
        dn2 = _msg_agg(x2, down2_index, Wdn[l], bdn[l], N2)
        x0 = _update(x0 + up0, Wu1[l], bu1[l], Wu2[l], bu2[l])
        x1 = _update(x1 + up1 + dn1, Wu1[l], bu1[l], Wu2[l], bu2[l])
        x2 = _update(x2 + dn2, Wu1[l], bu1[l], Wu2[l], bu2[l])
        f = jax.nn.sigmoid(jax.nn.relu(x0 @ Wf1[l] + bf1[l]) @ Wf2[l] + bf2[l])
        fmax = _seg_max(f, batch0, B)
        fmin = -_seg_max(-f, batch0, B)
        pers = jnp.concatenate([fmax, fmin], axis=-1)
        ph_vecs.append(jax.nn.relu(pers @ Wph[l] + bph[l]))
    pooled = (jax.ops.segment_sum(x0, batch0, num_segments=B)
              + jax.ops.segment_sum(x1, batch1, num_segments=B)
              + jax.ops.segment_sum(x2, batch2, num_segments=B))
    ph = jnp.stack(ph_vecs).mean(axis=0)
    out = pl.pallas_call(
        _head_kernel,
        out_shape=jax.ShapeDtypeStruct((B, NC_OUT), jnp.float32),
    )(pooled, ph, Wl1, bl1, Wl2, bl2)
    return out


# speedup vs baseline: 2.0885x; 1.1697x over previous
import functools

import jax
import jax.numpy as jnp
from jax import lax
from jax.experimental import pallas as pl
from jax.experimental.pallas import tpu as pltpu
from jax.experimental.pallas import tpu_sc as plsc

L = 3
N0, N1, N2 = 10000, 160000, 40000
E0, E1U, E1D, E2 = 320000, 480000, 320000, 480000
D = 128
B = 64
NCLS = 10

BLK = 2000
K = 128
B_E = 2000
CAP = 160


def _prefix_incl(m):
    lane = lax.iota(jnp.int32, 16)
    x = jnp.where(m, 1, 0)
    for sh in (1, 2, 4, 8):
        prev = jnp.take_along_axis(x, jnp.maximum(lane - sh, 0), axis=0)
        x = x + jnp.where(lane >= sh, prev, 0)
    return x


def _compact_idx(incl):
    lane = lax.iota(jnp.int32, 16)
    kp1 = lane + 1
    pos = jnp.zeros((16,), jnp.int32)
    for sh in (8, 4, 2, 1):
        val = jnp.take_along_axis(incl, pos + (sh - 1), axis=0)
        pos = jnp.where(val < kp1, pos + sh, pos)
    return pos


@functools.lru_cache(maxsize=None)
def _make_edge(E, CH, NCH):
    Epc = E // 16
    NB = Epc // B_E
    NV = B_E // 16
    CPS = NCH // 2
    CHT = CH // 16
    NPAD = NCH * CH + 8
    mesh = plsc.VectorSubcoreMesh(core_axis_name="c", subcore_axis_name="s",
                                  num_cores=2, num_subcores=16)

    @functools.partial(
        pl.kernel,
        out_type=jax.ShapeDtypeStruct((NCH * CH, D), jnp.float32),
        mesh=mesh,
        scratch_types=[
            pltpu.VMEM_SHARED((CH + 16, D), jnp.float32),
            pltpu.VMEM((64, D), jnp.float32),
            pltpu.VMEM((B_E,), jnp.int32),
            pltpu.VMEM((B_E,), jnp.int32),
            pltpu.VMEM((CAP,), jnp.int32),
            pltpu.VMEM((CAP,), jnp.int32),
            pltpu.VMEM((K,), jnp.int32),
            pltpu.VMEM((K,), jnp.int32),
            pltpu.VMEM((K,), jnp.int32),
            pltpu.VMEM((K, D), jnp.float32),
            pltpu.VMEM((K, D), jnp.float32),
            pltpu.SemaphoreType.DMA,
        ],
    )
    def ek(A, C, srce, dste, out, spm, zbuf, dstb, srcb, csrc, cdstl,
           gsrc, gdstg, gdstl, abuf, cbuf, sem):
        core = lax.axis_index("c")
        t = lax.axis_index("s")
        zv = jnp.zeros((16,), jnp.float32)

        def zero_row(r, carry):
            for j in range(8):
                zbuf[r, pl.ds(j * 16, 16)] = zv
            return carry
        lax.fori_loop(0, 64, zero_row, 0)

        def flush(lo):
            for j in range(8):
                sl = pl.ds(j * 16, 16)
                gsrc[sl] = csrc[sl]
                lv = cdstl[sl]
                gdstl[sl] = lv
                gdstg[sl] = lv + lo
            ca = pltpu.async_copy(A.at[gsrc], abuf, sem)
            cc = pltpu.async_copy(C.at[gdstg], cbuf, sem)
            ca.wait()
            cc.wait()

            def row(r, carry):
                for j in range(8):
                    sl = pl.ds(j * 16, 16)
                    abuf[r, sl] = jnp.maximum(abuf[r, sl] + cbuf[r, sl], 0.0)
                return carry
            lax.fori_loop(0, K, row, 0)
            pltpu.sync_copy(abuf, spm.at[gdstl], add=True)

        for ci in range(CPS):
            cid = ci * 2 + core
            lo = cid * CH
            for q in range(CHT // 64):
                pltpu.sync_copy(zbuf, spm.at[pl.ds(t * CHT + q * 64, 64)])
            plsc.subcore_barrier()

            def block_body(b, fill):
                base = t * Epc + b * B_E
                pltpu.sync_copy(dste.at[pl.ds(base, B_E)], dstb)
                pltpu.sync_copy(srce.at[pl.ds(base, B_E)], srcb)

                def vec_body(v, fill):
                    d16 = dstb[pl.ds(v * 16, 16)]
                    s16 = srcb[pl.ds(v * 16, 16)]
                    l16 = d16 - lo
                    inb = (l16 >= 0) & (l16 < CH)
                    incl = _prefix_incl(inb)
                    cnt = incl[15]
                    sidx = _compact_idx(incl)
                    csrc[pl.ds(fill, 16)] = jnp.take_along_axis(s16, sidx, axis=0)
                    cdstl[pl.ds(fill, 16)] = jnp.take_along_axis(l16, sidx, axis=0)
                    fill = fill + cnt

                    def do_flush():
                        flush(lo)
                        csrc[pl.ds(0, 16)] = csrc[pl.ds(K, 16)]
                        cdstl[pl.ds(0, 16)] = cdstl[pl.ds(K, 16)]
                        return fill - K

                    return lax.cond(fill >= K, do_flush, lambda: fill)

                return lax.fori_loop(0, NV, vec_body, fill)

            fill = lax.fori_loop(0, NB, block_body, 0)

            @pl.when(fill > 0)
            def _():
                padl = jnp.full((16,), CH, jnp.int32)
                pad0 = jnp.zeros((16,), jnp.int32)
                cdstl[pl.ds(fill, 16)] = padl
                csrc[pl.ds(fill, 16)] = pad0
                for j in range(1, 8):
                    @pl.when(j * 16 >= fill)
                    def _():
                        sl = pl.ds(j * 16, 16)
                        cdstl[sl] = padl
                        csrc[sl] = pad0
                flush(lo)

            plsc.subcore_barrier()
            pltpu.sync_copy(spm.at[pl.ds(t * CHT, CHT)],
                            out.at[pl.ds(lo + t * CHT, CHT)])
            plsc.subcore_barrier()

    return ek, NPAD


def _proj_body(x_ref, w_ref, b_ref, *out_refs):
    y = jnp.dot(x_ref[...], w_ref[...], preferred_element_type=jnp.float32)
    y = y + b_ref[...]
    for i, o in enumerate(out_refs):
        o[...] = y[:, i * D:(i + 1) * D]


def _proj(x, w, bvec, nout, npad):
    n = x.shape[0]
    grid = n // BLK
    return pl.pallas_call(
        _proj_body,
        grid=(grid,),
        in_specs=[
            pl.BlockSpec((BLK, D), lambda g: (g, 0)),
            pl.BlockSpec((D, nout * D), lambda g: (0, 0)),
            pl.BlockSpec((1, nout * D), lambda g: (0, 0)),
        ],
        out_specs=[pl.BlockSpec((BLK, D), lambda g: (g, 0))] * nout,
        out_shape=[jax.ShapeDtypeStruct((npad, D), jnp.float32)] * nout,
    )(x, w, bvec.reshape(1, -1))


def _upd_body(x_ref, u_ref, v_ref, w1_ref, b1_ref, w2_ref, b2_ref, o_ref):
    s = x_ref[...] + u_ref[...]
    if v_ref is not None:
        s = s + v_ref[...]
    h = jnp.maximum(jnp.dot(s, w1_ref[...],
                            preferred_element_type=jnp.float32) + b1_ref[...], 0.0)
    o_ref[...] = jnp.maximum(jnp.dot(h, w2_ref[...],
                                     preferred_element_type=jnp.float32)
                             + b2_ref[...], 0.0)


def _update(x, u, v, W1, b1, W2, b2):
    n = x.shape[0]
    grid = n // BLK
    xspec = pl.BlockSpec((BLK, D), lambda g: (g, 0))
    wspec = pl.BlockSpec((D, D), lambda g: (0, 0))
    bspec = pl.BlockSpec((1, D), lambda g: (0, 0))
    if v is None:
        body = lambda x_ref, u_ref, w1, b1_, w2, b2_, o: _upd_body(
            x_ref, u_ref, None, w1, b1_, w2, b2_, o)
        in_specs = [xspec, xspec, wspec, bspec, wspec, bspec]
        args = (x, u, W1, b1.reshape(1, D), W2, b2.reshape(1, D))
    else:
        body = _upd_body
        in_specs = [xspec, xspec, xspec, wspec, bspec, wspec, bspec]
        args = (x, u, v, W1, b1.reshape(1, D), W2, b2.reshape(1, D))
    return pl.pallas_call(
        body,
        grid=(grid,),
        in_specs=in_specs,
        out_specs=xspec,
        out_shape=jax.ShapeDtypeStruct((n, D), jnp.float32),
    )(*args)


def _fhead_body(x_ref, seg_ref, wf1_ref, bf1_ref, wf2_ref, bf2_ref,
                wph_ref, bph_ref, o_ref, pers_ref):
    h = jnp.maximum(jnp.dot(x_ref[...], wf1_ref[...],
                            preferred_element_type=jnp.float32) + bf1_ref[...],
                    0.0)
    f = jax.nn.sigmoid(jnp.dot(h, wf2_ref[...],
                               preferred_element_type=jnp.float32)
                       + bf2_ref[...])
    seg = seg_ref[...]

    def body(b, carry):
        msk = seg == b
        mx = jnp.max(jnp.where(msk, f, -1e30), axis=0)
        mn = jnp.max(jnp.where(msk, -f, -1e30), axis=0)
        mx = jnp.where(mx < -1e29, 0.0, mx)
        mn = jnp.where(mn < -1e29, 0.0, mn)
        pers_ref[pl.ds(b, 1), :] = jnp.concatenate([mx, -mn])[None, :]
        return carry
    lax.fori_loop(0, B, body, 0)
    o_ref[...] = jnp.maximum(jnp.dot(pers_ref[...], wph_ref[...],
                                     preferred_element_type=jnp.float32)
                             + bph_ref[...], 0.0)


def _fhead(x0, seg2d, Wf1, bf1, Wf2, bf2, Wph, bph):
    return pl.pallas_call(
        _fhead_body,
        out_shape=jax.ShapeDtypeStruct((B, 64), jnp.float32),
        scratch_shapes=[pltpu.VMEM((B, 16), jnp.float32)],
    )(x0, seg2d, Wf1, bf1.reshape(1, 16), Wf2, bf2.reshape(1, 8),
      Wph, bph.reshape(1, 64))


def _pool_body(x_ref, seg_ref, o_ref):
    @pl.when(pl.program_id(0) == 0)
    def _():
        o_ref[...] = jnp.zeros_like(o_ref)
    oh = (lax.broadcasted_iota(jnp.int32, (BLK, B), 1) == seg_ref[...]
          ).astype(jnp.float32)
    o_ref[...] += lax.dot_general(oh, x_ref[...], (((0,), (0,)), ((), ())),
                                  preferred_element_type=jnp.float32)


def _pool(x, seg2d):
    n = x.shape[0]
    return pl.pallas_call(
        _pool_body,
        grid=(n // BLK,),
        in_specs=[pl.BlockSpec((BLK, D), lambda g: (g, 0)),
                  pl.BlockSpec((BLK, 1), lambda g: (g, 0))],
        out_specs=pl.BlockSpec((B, D), lambda g: (0, 0)),
        out_shape=jax.ShapeDtypeStruct((B, D), jnp.float32),
    )(x, seg2d)


def _final_body(p0, p1, p2, h1, h2, h3, wl1, bl1, wl2, bl2, o_ref):
    pooled = p0[...] + p1[...] + p2[...]
    h = jnp.maximum(jnp.dot(pooled, wl1[...],
                            preferred_element_type=jnp.float32) + bl1[...], 0.0)
    ph = (h1[...] + h2[...] + h3[...]) * (1.0 / 3.0)
    cat = jnp.concatenate([h, ph], axis=1)
    o_ref[...] = jnp.dot(cat, wl2[...],
                         preferred_element_type=jnp.float32) + bl2[...]


def kernel(x0, x1, x2, up0_index, up1_index, down1_index, down2_index,
           batch0, batch1, batch2,
           Wup, bup, Wdn, bdn, Wu1, bu1, Wu2, bu2,
           Wf1, bf1, Wf2, bf2, Wph, bph, Wl1, bl1, Wl2, bl2):
    ek_up0, np0 = _make_edge(E0, 5120, 2)
    ek_up1, np1 = _make_edge(E1U, 8192, 20)
    ek_dn1, np1b = _make_edge(E1D, 8192, 20)
    ek_dn2, np2 = _make_edge(E2, 8192, 6)

    s0 = up0_index[0].astype(jnp.int32)
    d0 = up0_index[1].astype(jnp.int32)
    s1u = up1_index[0].astype(jnp.int32)
    d1u = up1_index[1].astype(jnp.int32)
    s1d = down1_index[0].astype(jnp.int32)
    d1d = down1_index[1].astype(jnp.int32)
    s2 = down2_index[0].astype(jnp.int32)
    d2 = down2_index[1].astype(jnp.int32)
    seg0 = batch0.astype(jnp.int32).reshape(N0, 1)
    seg1 = batch1.astype(jnp.int32).reshape(N1, 1)
    seg2 = batch2.astype(jnp.int32).reshape(N2, 1)
    z128 = jnp.zeros((D,), jnp.float32)

    phs = []
    for l in range(L):
        wu_cat = jnp.concatenate([Wup[l][:D], Wup[l][D:]], axis=1)
        wd_cat = jnp.concatenate([Wdn[l][:D], Wdn[l][D:]], axis=1)
        bu_cat = jnp.concatenate([z128, bup[l]])
        bd_cat = jnp.concatenate([z128, bdn[l]])
        w1_cat = jnp.concatenate([wu_cat, wd_cat], axis=1)
        b1_cat = jnp.concatenate([bu_cat, bd_cat])

        A0, C0 = _proj(x0, wu_cat, bu_cat, 2, np0)
        A1u, C1u, A1d, C1d = _proj(x1, w1_cat, b1_cat, 4, np1)
        A2, C2 = _proj(x2, wd_cat, bd_cat, 2, np2)

        up0 = ek_up0(A0, C0, s0, d0)[:N0]
        up1 = ek_up1(A1u, C1u, s1u, d1u)[:N1]
        dn1 = ek_dn1(A1d, C1d, s1d, d1d)[:N1]
        dn2 = ek_dn2(A2, C2, s2, d2)[:N2]

        x0 = _update(x0, up0, None, Wu1[l], bu1[l], Wu2[l], bu2[l])
        x1 = _update(x1, up1, dn1, Wu1[l], bu1[l], Wu2[l], bu2[l])
        x2 = _update(x2, dn2, None, Wu1[l], bu1[l], Wu2[l], bu2[l])

        phs.append(_fhead(x0, seg0, Wf1[l], bf1[l], Wf2[l], bf2[l],
                          Wph[l], bph[l]))

    p0 = _pool(x0, seg0)
    p1 = _pool(x1, seg1)
    p2 = _pool(x2, seg2)

    return pl.pallas_call(
        _final_body,
        out_shape=jax.ShapeDtypeStruct((B, NCLS), jnp.float32),
    )(p0, p1, p2, phs[0], phs[1], phs[2],
      Wl1, bl1.reshape(1, D), Wl2, bl2.reshape(1, NCLS))


# speedup vs baseline: 3.2721x; 1.5667x over previous
import functools

import jax
import jax.numpy as jnp
from jax import lax
from jax.experimental import pallas as pl
from jax.experimental.pallas import tpu as pltpu
from jax.experimental.pallas import tpu_sc as plsc

L = 3
N0, N1, N2 = 10000, 160000, 40000
E0, E1U, E1D, E2 = 320000, 480000, 320000, 480000
D = 128
B = 64
NCLS = 10

BLK = 2000
K = 64
B_E = 2000
CAP = 96


def _prefix_incl(m):
    lane = lax.iota(jnp.int32, 16)
    x = jnp.where(m, 1, 0)
    for sh in (1, 2, 4, 8):
        prev = jnp.take_along_axis(x, jnp.maximum(lane - sh, 0), axis=0)
        x = x + jnp.where(lane >= sh, prev, 0)
    return x


def _compact_idx(incl):
    lane = lax.iota(jnp.int32, 16)
    kp1 = lane + 1
    pos = jnp.zeros((16,), jnp.int32)
    for sh in (8, 4, 2, 1):
        val = jnp.take_along_axis(incl, pos + (sh - 1), axis=0)
        pos = jnp.where(val < kp1, pos + sh, pos)
    return pos


@functools.lru_cache(maxsize=None)
def _make_bucket(E, CH, NCH):
    Epc = E // 16
    NB = Epc // B_E
    NV = B_E // 16
    CPS = NCH // 2
    CAPE = Epc + K
    LL = NCH * 16 * CAPE
    mesh = plsc.VectorSubcoreMesh(core_axis_name="c", subcore_axis_name="s",
                                  num_cores=2, num_subcores=16)

    @functools.partial(
        pl.kernel,
        out_type=(jax.ShapeDtypeStruct((LL,), jnp.int32),
                  jax.ShapeDtypeStruct((LL,), jnp.int32),
                  jax.ShapeDtypeStruct((NCH * 16 * 16,), jnp.int32)),
        mesh=mesh,
        scratch_types=[
            pltpu.VMEM((B_E,), jnp.int32),
            pltpu.VMEM((B_E,), jnp.int32),
            pltpu.VMEM((CAP,), jnp.int32),
            pltpu.VMEM((CAP,), jnp.int32),
            pltpu.VMEM((16,), jnp.int32),
        ],
    )
    def bk(srce, dste, lsrc, ldstl, cnts, dstb, srcb, csrc, cdstl, nbuf):
        core = lax.axis_index("c")
        t = lax.axis_index("s")

        for ci in range(CPS):
            cid = ci * 2 + core
            lo = cid * CH
            rb = (cid * 16 + t) * CAPE

            def wblock(nb):
                pltpu.sync_copy(csrc.at[pl.ds(0, K)],
                                lsrc.at[pl.ds(rb + nb * K, K)])
                pltpu.sync_copy(cdstl.at[pl.ds(0, K)],
                                ldstl.at[pl.ds(rb + nb * K, K)])
                return nb + 1

            def block_body(b, carry):
                fill, nb = carry
                base = t * Epc + b * B_E
                pltpu.sync_copy(dste.at[pl.ds(base, B_E)], dstb)
                pltpu.sync_copy(srce.at[pl.ds(base, B_E)], srcb)

                def vec_body(v, carry):
                    fill, nb = carry
                    d16 = dstb[pl.ds(v * 16, 16)]
                    s16 = srcb[pl.ds(v * 16, 16)]
                    l16 = d16 - lo
                    inb = (l16 >= 0) & (l16 < CH)
                    incl = _prefix_incl(inb)
                    cnt = incl[15]
                    sidx = _compact_idx(incl)
                    csrc[pl.ds(fill, 16)] = jnp.take_along_axis(s16, sidx,
                                                                axis=0)
                    cdstl[pl.ds(fill, 16)] = jnp.take_along_axis(l16, sidx,
                                                                 axis=0)
                    fill = fill + cnt

                    def do_flush():
                        nb2 = wblock(nb)
                        csrc[pl.ds(0, 16)] = csrc[pl.ds(K, 16)]
                        cdstl[pl.ds(0, 16)] = cdstl[pl.ds(K, 16)]
                        return (fill - K, nb2)

                    return lax.cond(fill >= K, do_flush, lambda: (fill, nb))

                return lax.fori_loop(0, NV, vec_body, (fill, nb))

            fill, nb = lax.fori_loop(0, NB, block_body, (0, 0))

            def final_flush():
                padl = jnp.full((16,), CH, jnp.int32)
                pad0 = jnp.zeros((16,), jnp.int32)
                cdstl[pl.ds(fill, 16)] = padl
                csrc[pl.ds(fill, 16)] = pad0
                for j in range(1, K // 16):
                    @pl.when(j * 16 >= fill)
                    def _():
                        sl = pl.ds(j * 16, 16)
                        cdstl[sl] = padl
                        csrc[sl] = pad0
                return wblock(nb)

            nb = lax.cond(fill > 0, final_flush, lambda: nb)
            nbuf[pl.ds(0, 16)] = jnp.zeros((16,), jnp.int32) + nb
            pltpu.sync_copy(nbuf, cnts.at[pl.ds((cid * 16 + t) * 16, 16)])

    return bk, CAPE


@functools.lru_cache(maxsize=None)
def _make_edge(E, CH, NCH):
    Epc = E // 16
    CPS = NCH // 2
    CHT = CH // 16
    CAPE = Epc + K
    NPAD = NCH * CH + 8
    mesh = plsc.VectorSubcoreMesh(core_axis_name="c", subcore_axis_name="s",
                                  num_cores=2, num_subcores=16)

    @functools.partial(
        pl.kernel,
        out_type=jax.ShapeDtypeStruct((NCH * CH, D), jnp.float32),
        mesh=mesh,
        scratch_types=[
            pltpu.VMEM_SHARED((CH + 16, D), jnp.float32),
            pltpu.VMEM((16, D), jnp.float32),
            pltpu.VMEM((K,), jnp.int32),
            pltpu.VMEM((K,), jnp.int32),
            pltpu.VMEM((K,), jnp.int32),
            pltpu.VMEM((K,), jnp.int32),
            pltpu.VMEM((K,), jnp.int32),
            pltpu.VMEM((K,), jnp.int32),
            pltpu.VMEM((K, D), jnp.float32),
            pltpu.VMEM((K, D), jnp.float32),
            pltpu.VMEM((K, D), jnp.float32),
            pltpu.VMEM((K, D), jnp.float32),
            pltpu.VMEM((16,), jnp.int32),
            pltpu.SemaphoreType.DMA,
            pltpu.SemaphoreType.DMA,
        ],
    )
    def ek(A, C, lsrc, ldstl, cnts, out, spm, zbuf,
           gsrc0, gsrc1, gdstg0, gdstg1, gdstl0, gdstl1,
           abuf0, abuf1, cbuf0, cbuf1, nbuf, sem0, sem1):
        core = lax.axis_index("c")
        t = lax.axis_index("s")
        gsrc = (gsrc0, gsrc1)
        gdstg = (gdstg0, gdstg1)
        gdstl = (gdstl0, gdstl1)
        abuf = (abuf0, abuf1)
        cbuf = (cbuf0, cbuf1)
        sems = (sem0, sem1)
        zv = jnp.zeros((16,), jnp.float32)

        def zero_row(r, carry):
            for j in range(8):
                zbuf[r, pl.ds(j * 16, 16)] = zv
            return carry
        lax.fori_loop(0, 16, zero_row, 0)

        def stage(bi, pb, rb, lo):
            pltpu.sync_copy(lsrc.at[pl.ds(rb + bi * K, K)], gsrc[pb])
            pltpu.sync_copy(ldstl.at[pl.ds(rb + bi * K, K)], gdstl[pb])
            for j in range(K // 16):
                sl = pl.ds(j * 16, 16)
                gdstg[pb][sl] = gdstl[pb][sl] + lo
            pltpu.async_copy(A.at[gsrc[pb]], abuf[pb], sems[pb])
            pltpu.async_copy(C.at[gdstg[pb]], cbuf[pb], sems[pb])

        def drain_compute(pb):
            pltpu.make_async_copy(A.at[gsrc[pb]], abuf[pb], sems[pb]).wait()
            pltpu.make_async_copy(C.at[gdstg[pb]], cbuf[pb], sems[pb]).wait()

            def row(r, carry):
                for j in range(8):
                    sl = pl.ds(j * 16, 16)
                    abuf[pb][r, sl] = jnp.maximum(
                        abuf[pb][r, sl] + cbuf[pb][r, sl], 0.0)
                return carry
            lax.fori_loop(0, K, row, 0)
            pltpu.sync_copy(abuf[pb], spm.at[gdstl[pb]], add=True)

        for ci in range(CPS):
            cid = ci * 2 + core
            lo = cid * CH
            rb = (cid * 16 + t) * CAPE
            for q in range(CHT // 16):
                pltpu.sync_copy(zbuf, spm.at[pl.ds(t * CHT + q * 16, 16)])
            pltpu.sync_copy(cnts.at[pl.ds((cid * 16 + t) * 16, 16)], nbuf)
            plsc.subcore_barrier()
            nb = nbuf[pl.ds(0, 16)][0]

            @pl.when(nb > 0)
            def _():
                stage(0, 0, rb, lo)

                def blk_body(bi, carry):
                    def even():
                        @pl.when(bi + 1 < nb)
                        def _():
                            stage(bi + 1, 1, rb, lo)
                        drain_compute(0)
                        return 0

                    def odd():
                        @pl.when(bi + 1 < nb)
                        def _():
                            stage(bi + 1, 0, rb, lo)
                        drain_compute(1)
                        return 0

                    return lax.cond(bi % 2 == 0, even, odd)

                lax.fori_loop(0, nb, blk_body, 0)

            plsc.subcore_barrier()
            pltpu.sync_copy(spm.at[pl.ds(t * CHT, CHT)],
                            out.at[pl.ds(lo + t * CHT, CHT)])
            plsc.subcore_barrier()

    return ek, NPAD


def _proj_body(x_ref, w_ref, b_ref, *out_refs):
    y = jnp.dot(x_ref[...], w_ref[...], preferred_element_type=jnp.float32)
    y = y + b_ref[...]
    for i, o in enumerate(out_refs):
        o[...] = y[:, i * D:(i + 1) * D]


def _proj(x, w, bvec, nout, npad):
    n = x.shape[0]
    grid = n // BLK
    return pl.pallas_call(
        _proj_body,
        grid=(grid,),
        in_specs=[
            pl.BlockSpec((BLK, D), lambda g: (g, 0)),
            pl.BlockSpec((D, nout * D), lambda g: (0, 0)),
            pl.BlockSpec((1, nout * D), lambda g: (0, 0)),
        ],
        out_specs=[pl.BlockSpec((BLK, D), lambda g: (g, 0))] * nout,
        out_shape=[jax.ShapeDtypeStruct((npad, D), jnp.float32)] * nout,
    )(x, w, bvec.reshape(1, -1))


def _upd_body(x_ref, u_ref, v_ref, w1_ref, b1_ref, w2_ref, b2_ref, o_ref):
    s = x_ref[...] + u_ref[...]
    if v_ref is not None:
        s = s + v_ref[...]
    h = jnp.maximum(jnp.dot(s, w1_ref[...],
                            preferred_element_type=jnp.float32) + b1_ref[...], 0.0)
    o_ref[...] = jnp.maximum(jnp.dot(h, w2_ref[...],
                                     preferred_element_type=jnp.float32)
                             + b2_ref[...], 0.0)


def _update(x, u, v, W1, b1, W2, b2):
    n = x.shape[0]
    grid = n // BLK
    xspec = pl.BlockSpec((BLK, D), lambda g: (g, 0))
    wspec = pl.BlockSpec((D, D), lambda g: (0, 0))
    bspec = pl.BlockSpec((1, D), lambda g: (0, 0))
    if v is None:
        body = lambda x_ref, u_ref, w1, b1_, w2, b2_, o: _upd_body(
            x_ref, u_ref, None, w1, b1_, w2, b2_, o)
        in_specs = [xspec, xspec, wspec, bspec, wspec, bspec]
        args = (x, u, W1, b1.reshape(1, D), W2, b2.reshape(1, D))
    else:
        body = _upd_body
        in_specs = [xspec, xspec, xspec, wspec, bspec, wspec, bspec]
        args = (x, u, v, W1, b1.reshape(1, D), W2, b2.reshape(1, D))
    return pl.pallas_call(
        body,
        grid=(grid,),
        in_specs=in_specs,
        out_specs=xspec,
        out_shape=jax.ShapeDtypeStruct((n, D), jnp.float32),
    )(*args)


def _fhead_body(x_ref, seg_ref, wf1_ref, bf1_ref, wf2_ref, bf2_ref,
                wph_ref, bph_ref, o_ref, pers_ref):
    h = jnp.maximum(jnp.dot(x_ref[...], wf1_ref[...],
                            preferred_element_type=jnp.float32) + bf1_ref[...],
                    0.0)
    f = jax.nn.sigmoid(jnp.dot(h, wf2_ref[...],
                               preferred_element_type=jnp.float32)
                       + bf2_ref[...])
    seg = seg_ref[...]

    def body(b, carry):
        msk = seg == b
        mx = jnp.max(jnp.where(msk, f, -1e30), axis=0)
        mn = jnp.max(jnp.where(msk, -f, -1e30), axis=0)
        mx = jnp.where(mx < -1e29, 0.0, mx)
        mn = jnp.where(mn < -1e29, 0.0, mn)
        pers_ref[pl.ds(b, 1), :] = jnp.concatenate([mx, -mn])[None, :]
        return carry
    lax.fori_loop(0, B, body, 0)
    o_ref[...] = jnp.maximum(jnp.dot(pers_ref[...], wph_ref[...],
                                     preferred_element_type=jnp.float32)
                             + bph_ref[...], 0.0)


def _fhead(x0, seg2d, Wf1, bf1, Wf2, bf2, Wph, bph):
    return pl.pallas_call(
        _fhead_body,
        out_shape=jax.ShapeDtypeStruct((B, 64), jnp.float32),
        scratch_shapes=[pltpu.VMEM((B, 16), jnp.float32)],
    )(x0, seg2d, Wf1, bf1.reshape(1, 16), Wf2, bf2.reshape(1, 8),
      Wph, bph.reshape(1, 64))


def _pool_body(x_ref, seg_ref, o_ref):
    @pl.when(pl.program_id(0) == 0)
    def _():
        o_ref[...] = jnp.zeros_like(o_ref)
    oh = (lax.broadcasted_iota(jnp.int32, (BLK, B), 1) == seg_ref[...]
          ).astype(jnp.float32)
    o_ref[...] += lax.dot_general(oh, x_ref[...], (((0,), (0,)), ((), ())),
                                  preferred_element_type=jnp.float32)


def _pool(x, seg2d):
    n = x.shape[0]
    return pl.pallas_call(
        _pool_body,
        grid=(n // BLK,),
        in_specs=[pl.BlockSpec((BLK, D), lambda g: (g, 0)),
                  pl.BlockSpec((BLK, 1), lambda g: (g, 0))],
        out_specs=pl.BlockSpec((B, D), lambda g: (0, 0)),
        out_shape=jax.ShapeDtypeStruct((B, D), jnp.float32),
    )(x, seg2d)


def _final_body(p0, p1, p2, h1, h2, h3, wl1, bl1, wl2, bl2, o_ref):
    pooled = p0[...] + p1[...] + p2[...]
    h = jnp.maximum(jnp.dot(pooled, wl1[...],
                            preferred_element_type=jnp.float32) + bl1[...], 0.0)
    ph = (h1[...] + h2[...] + h3[...]) * (1.0 / 3.0)
    cat = jnp.concatenate([h, ph], axis=1)
    o_ref[...] = jnp.dot(cat, wl2[...],
                         preferred_element_type=jnp.float32) + bl2[...]


def kernel(x0, x1, x2, up0_index, up1_index, down1_index, down2_index,
           batch0, batch1, batch2,
           Wup, bup, Wdn, bdn, Wu1, bu1, Wu2, bu2,
           Wf1, bf1, Wf2, bf2, Wph, bph, Wl1, bl1, Wl2, bl2):
    ek_up0, np0 = _make_edge(E0, 5120, 2)
    ek_up1, np1 = _make_edge(E1U, 8192, 20)
    ek_dn1, np1b = _make_edge(E1D, 8192, 20)
    ek_dn2, np2 = _make_edge(E2, 8192, 6)
    bk_up0, _ = _make_bucket(E0, 5120, 2)
    bk_up1, _ = _make_bucket(E1U, 8192, 20)
    bk_dn1, _ = _make_bucket(E1D, 8192, 20)
    bk_dn2, _ = _make_bucket(E2, 8192, 6)

    s0 = up0_index[0].astype(jnp.int32)
    d0 = up0_index[1].astype(jnp.int32)
    s1u = up1_index[0].astype(jnp.int32)
    d1u = up1_index[1].astype(jnp.int32)
    s1d = down1_index[0].astype(jnp.int32)
    d1d = down1_index[1].astype(jnp.int32)
    s2 = down2_index[0].astype(jnp.int32)
    d2 = down2_index[1].astype(jnp.int32)

    lists = {
        'up0': bk_up0(s0, d0),
        'up1': bk_up1(s1u, d1u),
        'dn1': bk_dn1(s1d, d1d),
        'dn2': bk_dn2(s2, d2),
    }
    seg0 = batch0.astype(jnp.int32).reshape(N0, 1)
    seg1 = batch1.astype(jnp.int32).reshape(N1, 1)
    seg2 = batch2.astype(jnp.int32).reshape(N2, 1)
    z128 = jnp.zeros((D,), jnp.float32)

    phs = []
    for l in range(L):
        wu_cat = jnp.concatenate([Wup[l][:D], Wup[l][D:]], axis=1)
        wd_cat = jnp.concatenate([Wdn[l][:D], Wdn[l][D:]], axis=1)
        bu_cat = jnp.concatenate([z128, bup[l]])
        bd_cat = jnp.concatenate([z128, bdn[l]])
        w1_cat = jnp.concatenate([wu_cat, wd_cat], axis=1)
        b1_cat = jnp.concatenate([bu_cat, bd_cat])

        A0, C0 = _proj(x0, wu_cat, bu_cat, 2, np0)
        A1u, C1u, A1d, C1d = _proj(x1, w1_cat, b1_cat, 4, np1)
        A2, C2 = _proj(x2, wd_cat, bd_cat, 2, np2)

        up0 = ek_up0(A0, C0, *lists['up0'])[:N0]
        up1 = ek_up1(A1u, C1u, *lists['up1'])[:N1]
        dn1 = ek_dn1(A1d, C1d, *lists['dn1'])[:N1]
        dn2 = ek_dn2(A2, C2, *lists['dn2'])[:N2]

        x0 = _update(x0, up0, None, Wu1[l], bu1[l], Wu2[l], bu2[l])
        x1 = _update(x1, up1, dn1, Wu1[l], bu1[l], Wu2[l], bu2[l])
        x2 = _update(x2, dn2, None, Wu1[l], bu1[l], Wu2[l], bu2[l])

        phs.append(_fhead(x0, seg0, Wf1[l], bf1[l], Wf2[l], bf2[l],
                          Wph[l], bph[l]))

    p0 = _pool(x0, seg0)
    p1 = _pool(x1, seg1)
    p2 = _pool(x2, seg2)

    return pl.pallas_call(
        _final_body,
        out_shape=jax.ShapeDtypeStruct((B, NCLS), jnp.float32),
    )(p0, p1, p2, phs[0], phs[1], phs[2],
      Wl1, bl1.reshape(1, D), Wl2, bl2.reshape(1, NCLS))


# speedup vs baseline: 3.5552x; 1.0865x over previous
import functools

import jax
import jax.numpy as jnp
from jax import lax
from jax.experimental import pallas as pl
from jax.experimental.pallas import tpu as pltpu
from jax.experimental.pallas import tpu_sc as plsc

L = 3
N0, N1, N2 = 10000, 160000, 40000
E0, E1U, E1D, E2 = 320000, 480000, 320000, 480000
D = 128
B = 64
NCLS = 10

BLK = 2000
K = 64
IB = 8
B_E = 2000
CAP = 96


def _prefix_incl(m):
    lane = lax.iota(jnp.int32, 16)
    x = jnp.where(m, 1, 0)
    for sh in (1, 2, 4, 8):
        prev = jnp.take_along_axis(x, jnp.maximum(lane - sh, 0), axis=0)
        x = x + jnp.where(lane >= sh, prev, 0)
    return x


def _compact_idx(incl):
    lane = lax.iota(jnp.int32, 16)
    kp1 = lane + 1
    pos = jnp.zeros((16,), jnp.int32)
    for sh in (8, 4, 2, 1):
        val = jnp.take_along_axis(incl, pos + (sh - 1), axis=0)
        pos = jnp.where(val < kp1, pos + sh, pos)
    return pos


@functools.lru_cache(maxsize=None)
def _make_bucket(E, CH, NCH):
    Epc = E // 16
    NB = Epc // B_E
    NV = B_E // 16
    CPS = NCH // 2
    CAPE = Epc + IB * K
    LL = NCH * 16 * CAPE
    mesh = plsc.VectorSubcoreMesh(core_axis_name="c", subcore_axis_name="s",
                                  num_cores=2, num_subcores=16)

    @functools.partial(
        pl.kernel,
        out_type=(jax.ShapeDtypeStruct((LL,), jnp.int32),
                  jax.ShapeDtypeStruct((LL,), jnp.int32),
                  jax.ShapeDtypeStruct((NCH * 16 * 16,), jnp.int32)),
        mesh=mesh,
        scratch_types=[
            pltpu.VMEM((B_E,), jnp.int32),
            pltpu.VMEM((B_E,), jnp.int32),
            pltpu.VMEM((CAP,), jnp.int32),
            pltpu.VMEM((CAP,), jnp.int32),
            pltpu.VMEM((16,), jnp.int32),
        ],
    )
    def bk(srce, dste, lsrc, ldstl, cnts, dstb, srcb, csrc, cdstl, nbuf):
        core = lax.axis_index("c")
        t = lax.axis_index("s")

        for ci in range(CPS):
            cid = ci * 2 + core
            lo = cid * CH
            rb = (cid * 16 + t) * CAPE

            def wblock(nb):
                pltpu.sync_copy(csrc.at[pl.ds(0, K)],
                                lsrc.at[pl.ds(rb + nb * K, K)])
                pltpu.sync_copy(cdstl.at[pl.ds(0, K)],
                                ldstl.at[pl.ds(rb + nb * K, K)])
                return nb + 1

            def block_body(b, carry):
                fill, nb = carry
                base = t * Epc + b * B_E
                pltpu.sync_copy(dste.at[pl.ds(base, B_E)], dstb)
                pltpu.sync_copy(srce.at[pl.ds(base, B_E)], srcb)

                def vec_body(v, carry):
                    fill, nb = carry
                    d16 = dstb[pl.ds(v * 16, 16)]
                    s16 = srcb[pl.ds(v * 16, 16)]
                    l16 = d16 - lo
                    inb = (l16 >= 0) & (l16 < CH)
                    incl = _prefix_incl(inb)
                    cnt = incl[15]
                    sidx = _compact_idx(incl)
                    csrc[pl.ds(fill, 16)] = jnp.take_along_axis(s16, sidx,
                                                                axis=0)
                    cdstl[pl.ds(fill, 16)] = jnp.take_along_axis(l16, sidx,
                                                                 axis=0)
                    fill = fill + cnt

                    def do_flush():
                        nb2 = wblock(nb)
                        csrc[pl.ds(0, 16)] = csrc[pl.ds(K, 16)]
                        cdstl[pl.ds(0, 16)] = cdstl[pl.ds(K, 16)]
                        return (fill - K, nb2)

                    return lax.cond(fill >= K, do_flush, lambda: (fill, nb))

                return lax.fori_loop(0, NV, vec_body, (fill, nb))

            fill, nb = lax.fori_loop(0, NB, block_body, (0, 0))

            def final_flush():
                padl = jnp.full((16,), CH, jnp.int32)
                pad0 = jnp.zeros((16,), jnp.int32)
                cdstl[pl.ds(fill, 16)] = padl
                csrc[pl.ds(fill, 16)] = pad0
                for j in range(1, K // 16):
                    @pl.when(j * 16 >= fill)
                    def _():
                        sl = pl.ds(j * 16, 16)
                        cdstl[sl] = padl
                        csrc[sl] = pad0
                return wblock(nb)

            nb = lax.cond(fill > 0, final_flush, lambda: nb)
            nbuf[pl.ds(0, 16)] = jnp.zeros((16,), jnp.int32) + nb
            pltpu.sync_copy(nbuf, cnts.at[pl.ds((cid * 16 + t) * 16, 16)])

    return bk, CAPE


@functools.lru_cache(maxsize=None)
def _make_edge(E, CH, NCH):
    Epc = E // 16
    CPS = NCH // 2
    CHT = CH // 16
    CAPE = Epc + IB * K
    NPAD = NCH * CH + 8
    mesh = plsc.VectorSubcoreMesh(core_axis_name="c", subcore_axis_name="s",
                                  num_cores=2, num_subcores=16)

    @functools.partial(
        pl.kernel,
        out_type=jax.ShapeDtypeStruct((NCH * CH, D), jnp.float32),
        mesh=mesh,
        scratch_types=[
            pltpu.VMEM_SHARED((CH + 16, D), jnp.float32),
            pltpu.VMEM((IB * K,), jnp.int32),
            pltpu.VMEM((IB * K,), jnp.int32),
            pltpu.VMEM((K,), jnp.int32),
            pltpu.VMEM((K,), jnp.int32),
            pltpu.VMEM((K,), jnp.int32),
            pltpu.VMEM((K,), jnp.int32),
            pltpu.VMEM((K,), jnp.int32),
            pltpu.VMEM((K,), jnp.int32),
            pltpu.VMEM((K, D), jnp.float32),
            pltpu.VMEM((K, D), jnp.float32),
            pltpu.VMEM((K, D), jnp.float32),
            pltpu.VMEM((K, D), jnp.float32),
            pltpu.VMEM((16,), jnp.int32),
            pltpu.SemaphoreType.DMA,
            pltpu.SemaphoreType.DMA,
            pltpu.SemaphoreType.DMA,
            pltpu.SemaphoreType.DMA,
        ],
    )
    def ek(A, C, lsrc, ldstl, cnts, zrows, out, spm, gsrcB, gdstlB,
           gsrc0, gsrc1, gdstg0, gdstg1, gdstl0, gdstl1,
           abuf0, abuf1, cbuf0, cbuf1, nbuf, sem0, sem1, asem0, asem1):
        core = lax.axis_index("c")
        t = lax.axis_index("s")
        gsrc = (gsrc0, gsrc1)
        gdstg = (gdstg0, gdstg1)
        gdstl = (gdstl0, gdstl1)
        abuf = (abuf0, abuf1)
        cbuf = (cbuf0, cbuf1)
        sems = (sem0, sem1)
        asems = (asem0, asem1)

        def load_group(bi, rb):
            pltpu.sync_copy(lsrc.at[pl.ds(rb + bi * K, IB * K)], gsrcB)
            pltpu.sync_copy(ldstl.at[pl.ds(rb + bi * K, IB * K)], gdstlB)

        def wait_scatter(pb):
            pltpu.make_async_copy(abuf[pb], spm.at[gdstl[pb]],
                                  asems[pb]).wait()

        def stage(bi, pb, lo):
            p = bi % IB
            for j in range(K // 16):
                sl = pl.ds(j * 16, 16)
                sv = gsrcB[pl.ds(p * K + j * 16, 16)]
                lv = gdstlB[pl.ds(p * K + j * 16, 16)]
                gsrc[pb][sl] = sv
                gdstl[pb][sl] = lv
                gdstg[pb][sl] = lv + lo
            pltpu.async_copy(A.at[gsrc[pb]], abuf[pb], sems[pb])
            pltpu.async_copy(C.at[gdstg[pb]], cbuf[pb], sems[pb])

        def drain_compute(pb):
            pltpu.make_async_copy(A.at[gsrc[pb]], abuf[pb], sems[pb]).wait()
            pltpu.make_async_copy(C.at[gdstg[pb]], cbuf[pb], sems[pb]).wait()

            def row(r8, carry):
                for r0 in range(8):
                    for j in range(8):
                        sl = pl.ds(j * 16, 16)
                        a = abuf[pb]
                        c = cbuf[pb]
                        a[r8 * 8 + r0, sl] = jnp.maximum(
                            a[r8 * 8 + r0, sl] + c[r8 * 8 + r0, sl], 0.0)
                return carry
            lax.fori_loop(0, K // 8, row, 0)
            pltpu.async_copy(abuf[pb], spm.at[gdstl[pb]], add=True,
                             sem=asems[pb])

        for ci in range(CPS):
            cid = ci * 2 + core
            lo = cid * CH
            rb = (cid * 16 + t) * CAPE
            pltpu.sync_copy(zrows.at[pl.ds(0, CHT)],
                            spm.at[pl.ds(t * CHT, CHT)])
            pltpu.sync_copy(cnts.at[pl.ds((cid * 16 + t) * 16, 16)], nbuf)
            plsc.subcore_barrier()
            nb = nbuf[pl.ds(0, 16)][0]

            @pl.when(nb > 0)
            def _():
                load_group(0, rb)
                stage(0, 0, lo)

                def blk_body(bi, carry):
                    def halfstep(pb):
                        nxt = bi + 1

                        @pl.when(nxt < nb)
                        def _():
                            @pl.when(nxt % IB == 0)
                            def _():
                                load_group(nxt, rb)

                            @pl.when(bi >= 1)
                            def _():
                                wait_scatter(1 - pb)
                            stage(nxt, 1 - pb, lo)
                        drain_compute(pb)
                        return 0

                    return lax.cond(bi % 2 == 0,
                                    lambda: halfstep(0), lambda: halfstep(1))

                lax.fori_loop(0, nb, blk_body, 0)

                def _w0():
                    wait_scatter(0)
                    return 0

                def _w1():
                    wait_scatter(1)
                    return 0

                @pl.when(nb >= 1)
                def _():
                    lax.cond((nb - 1) % 2 == 0, _w0, _w1)

                @pl.when(nb >= 2)
                def _():
                    lax.cond(nb % 2 == 0, _w0, _w1)

            plsc.subcore_barrier()
            pltpu.sync_copy(spm.at[pl.ds(t * CHT, CHT)],
                            out.at[pl.ds(lo + t * CHT, CHT)])
            plsc.subcore_barrier()

    return ek, NPAD


def _proj_body(x_ref, w_ref, b_ref, *out_refs):
    y = jnp.dot(x_ref[...], w_ref[...], preferred_element_type=jnp.float32)
    y = y + b_ref[...]
    for i, o in enumerate(out_refs):
        o[...] = y[:, i * D:(i + 1) * D]


def _proj(x, w, bvec, nout, npad):
    n = x.shape[0]
    grid = n // BLK
    return pl.pallas_call(
        _proj_body,
        grid=(grid,),
        in_specs=[
            pl.BlockSpec((BLK, D), lambda g: (g, 0)),
            pl.BlockSpec((D, nout * D), lambda g: (0, 0)),
            pl.BlockSpec((1, nout * D), lambda g: (0, 0)),
        ],
        out_specs=[pl.BlockSpec((BLK, D), lambda g: (g, 0))] * nout,
        out_shape=[jax.ShapeDtypeStruct((npad, D), jnp.float32)] * nout,
    )(x, w, bvec.reshape(1, -1))


def _upd_body(x_ref, u_ref, v_ref, w1_ref, b1_ref, w2_ref, b2_ref, o_ref):
    s = x_ref[...] + u_ref[...]
    if v_ref is not None:
        s = s + v_ref[...]
    h = jnp.maximum(jnp.dot(s, w1_ref[...],
                            preferred_element_type=jnp.float32) + b1_ref[...], 0.0)
    o_ref[...] = jnp.maximum(jnp.dot(h, w2_ref[...],
                                     preferred_element_type=jnp.float32)
                             + b2_ref[...], 0.0)


def _update(x, u, v, W1, b1, W2, b2):
    n = x.shape[0]
    grid = n // BLK
    xspec = pl.BlockSpec((BLK, D), lambda g: (g, 0))
    wspec = pl.BlockSpec((D, D), lambda g: (0, 0))
    bspec = pl.BlockSpec((1, D), lambda g: (0, 0))
    if v is None:
        body = lambda x_ref, u_ref, w1, b1_, w2, b2_, o: _upd_body(
            x_ref, u_ref, None, w1, b1_, w2, b2_, o)
        in_specs = [xspec, xspec, wspec, bspec, wspec, bspec]
        args = (x, u, W1, b1.reshape(1, D), W2, b2.reshape(1, D))
    else:
        body = _upd_body
        in_specs = [xspec, xspec, xspec, wspec, bspec, wspec, bspec]
        args = (x, u, v, W1, b1.reshape(1, D), W2, b2.reshape(1, D))
    return pl.pallas_call(
        body,
        grid=(grid,),
        in_specs=in_specs,
        out_specs=xspec,
        out_shape=jax.ShapeDtypeStruct((n, D), jnp.float32),
    )(*args)


def _fhead_body(x_ref, seg_ref, wf1_ref, bf1_ref, wf2_ref, bf2_ref,
                wph_ref, bph_ref, o_ref, pers_ref):
    h = jnp.maximum(jnp.dot(x_ref[...], wf1_ref[...],
                            preferred_element_type=jnp.float32) + bf1_ref[...],
                    0.0)
    f = jax.nn.sigmoid(jnp.dot(h, wf2_ref[...],
                               preferred_element_type=jnp.float32)
                       + bf2_ref[...])
    seg = seg_ref[...]

    def body(b, carry):
        msk = seg == b
        mx = jnp.max(jnp.where(msk, f, -1e30), axis=0)
        mn = jnp.max(jnp.where(msk, -f, -1e30), axis=0)
        mx = jnp.where(mx < -1e29, 0.0, mx)
        mn = jnp.where(mn < -1e29, 0.0, mn)
        pers_ref[pl.ds(b, 1), :] = jnp.concatenate([mx, -mn])[None, :]
        return carry
    lax.fori_loop(0, B, body, 0)
    o_ref[...] = jnp.maximum(jnp.dot(pers_ref[...], wph_ref[...],
                                     preferred_element_type=jnp.float32)
                             + bph_ref[...], 0.0)


def _fhead(x0, seg2d, Wf1, bf1, Wf2, bf2, Wph, bph):
    return pl.pallas_call(
        _fhead_body,
        out_shape=jax.ShapeDtypeStruct((B, 64), jnp.float32),
        scratch_shapes=[pltpu.VMEM((B, 16), jnp.float32)],
    )(x0, seg2d, Wf1, bf1.reshape(1, 16), Wf2, bf2.reshape(1, 8),
      Wph, bph.reshape(1, 64))


def _pool_body(x_ref, seg_ref, o_ref):
    @pl.when(pl.program_id(0) == 0)
    def _():
        o_ref[...] = jnp.zeros_like(o_ref)
    oh = (lax.broadcasted_iota(jnp.int32, (BLK, B), 1) == seg_ref[...]
          ).astype(jnp.float32)
    o_ref[...] += lax.dot_general(oh, x_ref[...], (((0,), (0,)), ((), ())),
                                  preferred_element_type=jnp.float32)


def _pool(x, seg2d):
    n = x.shape[0]
    return pl.pallas_call(
        _pool_body,
        grid=(n // BLK,),
        in_specs=[pl.BlockSpec((BLK, D), lambda g: (g, 0)),
                  pl.BlockSpec((BLK, 1), lambda g: (g, 0))],
        out_specs=pl.BlockSpec((B, D), lambda g: (0, 0)),
        out_shape=jax.ShapeDtypeStruct((B, D), jnp.float32),
    )(x, seg2d)


def _final_body(p0, p1, p2, h1, h2, h3, wl1, bl1, wl2, bl2, o_ref):
    pooled = p0[...] + p1[...] + p2[...]
    h = jnp.maximum(jnp.dot(pooled, wl1[...],
                            preferred_element_type=jnp.float32) + bl1[...], 0.0)
    ph = (h1[...] + h2[...] + h3[...]) * (1.0 / 3.0)
    cat = jnp.concatenate([h, ph], axis=1)
    o_ref[...] = jnp.dot(cat, wl2[...],
                         preferred_element_type=jnp.float32) + bl2[...]


def kernel(x0, x1, x2, up0_index, up1_index, down1_index, down2_index,
           batch0, batch1, batch2,
           Wup, bup, Wdn, bdn, Wu1, bu1, Wu2, bu2,
           Wf1, bf1, Wf2, bf2, Wph, bph, Wl1, bl1, Wl2, bl2):
    ek_up0, np0 = _make_edge(E0, 5120, 2)
    ek_up1, np1 = _make_edge(E1U, 8192, 20)
    ek_dn1, np1b = _make_edge(E1D, 8192, 20)
    ek_dn2, np2 = _make_edge(E2, 8192, 6)
    bk_up0, _ = _make_bucket(E0, 5120, 2)
    bk_up1, _ = _make_bucket(E1U, 8192, 20)
    bk_dn1, _ = _make_bucket(E1D, 8192, 20)
    bk_dn2, _ = _make_bucket(E2, 8192, 6)

    s0 = up0_index[0].astype(jnp.int32)
    d0 = up0_index[1].astype(jnp.int32)
    s1u = up1_index[0].astype(jnp.int32)
    d1u = up1_index[1].astype(jnp.int32)
    s1d = down1_index[0].astype(jnp.int32)
    d1d = down1_index[1].astype(jnp.int32)
    s2 = down2_index[0].astype(jnp.int32)
    d2 = down2_index[1].astype(jnp.int32)

    lists = {
        'up0': bk_up0(s0, d0),
        'up1': bk_up1(s1u, d1u),
        'dn1': bk_dn1(s1d, d1d),
        'dn2': bk_dn2(s2, d2),
    }
    seg0 = batch0.astype(jnp.int32).reshape(N0, 1)
    seg1 = batch1.astype(jnp.int32).reshape(N1, 1)
    seg2 = batch2.astype(jnp.int32).reshape(N2, 1)
    z128 = jnp.zeros((D,), jnp.float32)
    z512 = jnp.zeros((512, D), jnp.float32)

    phs = []
    for l in range(L):
        wu_cat = jnp.concatenate([Wup[l][:D], Wup[l][D:]], axis=1)
        wd_cat = jnp.concatenate([Wdn[l][:D], Wdn[l][D:]], axis=1)
        bu_cat = jnp.concatenate([z128, bup[l]])
        bd_cat = jnp.concatenate([z128, bdn[l]])
        w1_cat = jnp.concatenate([wu_cat, wd_cat], axis=1)
        b1_cat = jnp.concatenate([bu_cat, bd_cat])

        A0, C0 = _proj(x0, wu_cat, bu_cat, 2, np0)
        A1u, C1u, A1d, C1d = _proj(x1, w1_cat, b1_cat, 4, np1)
        A2, C2 = _proj(x2, wd_cat, bd_cat, 2, np2)

        up0 = ek_up0(A0, C0, *lists['up0'], z512)[:N0]
        up1 = ek_up1(A1u, C1u, *lists['up1'], z512)[:N1]
        dn1 = ek_dn1(A1d, C1d, *lists['dn1'], z512)[:N1]
        dn2 = ek_dn2(A2, C2, *lists['dn2'], z512)[:N2]

        x0 = _update(x0, up0, None, Wu1[l], bu1[l], Wu2[l], bu2[l])
        x1 = _update(x1, up1, dn1, Wu1[l], bu1[l], Wu2[l], bu2[l])
        x2 = _update(x2, dn2, None, Wu1[l], bu1[l], Wu2[l], bu2[l])

        phs.append(_fhead(x0, seg0, Wf1[l], bf1[l], Wf2[l], bf2[l],
                          Wph[l], bph[l]))

    p0 = _pool(x0, seg0)
    p1 = _pool(x1, seg1)
    p2 = _pool(x2, seg2)

    return pl.pallas_call(
        _final_body,
        out_shape=jax.ShapeDtypeStruct((B, NCLS), jnp.float32),
    )(p0, p1, p2, phs[0], phs[1], phs[2],
      Wl1, bl1.reshape(1, D), Wl2, bl2.reshape(1, NCLS))
